# baseline (speedup 1.0000x reference)
import functools

import jax
import jax.numpy as jnp
from jax import lax
from jax.experimental import pallas as pl
from jax.experimental.pallas import tpu as pltpu
from jax.experimental.pallas import tpu_sc as plsc

VOCAB = 100000
EMBED_DIM = 16
BATCH = 1024
CTX = 20

VT = 2048
NBUF = 4
NSTEP = (VOCAB + VT - 1) // VT
LAST = VOCAB - (NSTEP - 1) * VT

NC = 2
NS = 16
NW = NC * NS
B_PER_W = BATCH // NW
IDX_PER_W = B_PER_W * CTX
IDX_CHUNK = 128
N_CHUNKS = IDX_PER_W // IDX_CHUNK


def _pool_sc(idx3, table):
    mesh = plsc.VectorSubcoreMesh(core_axis_name="c", subcore_axis_name="s")

    @functools.partial(
        pl.kernel,
        mesh=mesh,
        out_type=jax.ShapeDtypeStruct((BATCH, EMBED_DIM), jnp.float32),
        scratch_types=[
            pltpu.VMEM((N_CHUNKS, IDX_CHUNK), jnp.int32),
            pltpu.VMEM((IDX_PER_W, EMBED_DIM), jnp.float32),
            pltpu.VMEM((B_PER_W, EMBED_DIM), jnp.float32),
            pltpu.SemaphoreType.DMA,
        ],
        compiler_params=pltpu.CompilerParams(use_tc_tiling_on_sc=False),
    )
    def k(idx_hbm, table_hbm, out_hbm, idx_v, rows_v, pooled_v, sem):
        wid = lax.axis_index("s") * NC + lax.axis_index("c")
        pltpu.sync_copy(idx_hbm.at[wid], idx_v)
        copies = [
            pltpu.async_copy(
                table_hbm.at[idx_v.at[j]],
                rows_v.at[pl.ds(j * IDX_CHUNK, IDX_CHUNK)],
                sem,
            )
            for j in range(N_CHUNKS)
        ]
        for c in copies:
            c.wait()

        def body(b, carry):
            r0 = b * CTX
            acc = rows_v[r0]
            for j in range(1, CTX):
                acc = acc + rows_v[r0 + j]
            pooled_v[b] = acc
            return carry

        lax.fori_loop(0, B_PER_W, body, 0)
        pltpu.sync_copy(pooled_v, out_hbm.at[pl.ds(wid * B_PER_W, B_PER_W)])

    return k(idx3, table)


def _mm_body(wt_ref, b_ref, xt_ref, o_hbm, scratch, sems):
    i = pl.program_id(0)
    slot = lax.rem(i, NBUF)

    @pl.when(i >= NBUF)
    def _wait_prev():
        j = i - NBUF
        pltpu.make_async_copy(
            scratch.at[slot], o_hbm.at[pl.ds(j * VT, VT)], sems.at[slot]
        ).wait()

    lhs = jnp.concatenate([wt_ref[...], b_ref[...]], axis=0)
    scratch[slot] = lax.dot_general(
        lhs, xt_ref[...],
        dimension_numbers=(((0,), (0,)), ((), ())),
        preferred_element_type=jnp.float32,
    )

    @pl.when(i < NSTEP - 1)
    def _start_full():
        pltpu.make_async_copy(
            scratch.at[slot], o_hbm.at[pl.ds(i * VT, VT)], sems.at[slot]
        ).start()

    @pl.when(i == NSTEP - 1)
    def _start_last():
        pltpu.make_async_copy(
            scratch.at[slot, pl.ds(0, LAST)],
            o_hbm.at[pl.ds(i * VT, LAST)],
            sems.at[slot],
        ).start()

    @pl.when(i == NSTEP - 1)
    def _drain():
        for k in range(NBUF - 1):
            j = i - (NBUF - 1) + k
            pltpu.make_async_copy(
                scratch.at[lax.rem(j, NBUF)],
                o_hbm.at[pl.ds(j * VT, VT)],
                sems.at[lax.rem(j, NBUF)],
            ).wait()
        pltpu.make_async_copy(
            scratch.at[slot, pl.ds(0, LAST)],
            o_hbm.at[pl.ds(i * VT, LAST)],
            sems.at[slot],
        ).wait()


def _project_tc(Wt, b2, xt_aug):
    out_t = pl.pallas_call(
        _mm_body,
        grid=(NSTEP,),
        in_specs=[
            pl.BlockSpec((EMBED_DIM, VT), lambda i: (0, i)),
            pl.BlockSpec((1, VT), lambda i: (0, i)),
            pl.BlockSpec((EMBED_DIM + 1, BATCH), lambda i: (0, 0)),
        ],
        out_specs=pl.BlockSpec(memory_space=pl.ANY),
        out_shape=jax.ShapeDtypeStruct((VOCAB, BATCH), jnp.float32),
        scratch_shapes=[
            pltpu.VMEM((NBUF, VT, BATCH), jnp.float32),
            pltpu.SemaphoreType.DMA((NBUF,)),
        ],
        compiler_params=pltpu.CompilerParams(
            vmem_limit_bytes=100 * 1024 * 1024,
        ),
    )(Wt, b2, xt_aug)
    return out_t


def kernel(inputs, embed_table, W, b):
    idx3 = inputs.astype(jnp.int32).reshape(NW, N_CHUNKS, IDX_CHUNK)
    pooled = _pool_sc(idx3, embed_table)
    xt_aug = jnp.concatenate(
        [pooled.T, jnp.ones((1, BATCH), jnp.float32)], axis=0
    )
    out_t = _project_tc(W.T, b.reshape(1, VOCAB), xt_aug)
    return out_t.T


# speedup vs baseline: 1.1248x; 1.1248x over previous
import functools

import jax
import jax.numpy as jnp
from jax import lax
from jax.experimental import pallas as pl
from jax.experimental.pallas import tpu as pltpu
from jax.experimental.pallas import tpu_sc as plsc

VOCAB = 100000
EMBED_DIM = 16
BATCH = 1024
CTX = 20

VT = 2048
NBUF = 4
NSTEP = (VOCAB + VT - 1) // VT
LAST = VOCAB - (NSTEP - 1) * VT

NC = 2
NS = 16
NW = NC * NS
B_PER_W = BATCH // NW
IDX_PER_W = B_PER_W * CTX
IDX_CHUNK = 128
N_CHUNKS = IDX_PER_W // IDX_CHUNK


def _pool_sc_t(idx3, table_t):
    mesh = plsc.VectorSubcoreMesh(core_axis_name="c", subcore_axis_name="s")

    @functools.partial(
        pl.kernel,
        mesh=mesh,
        out_type=jax.ShapeDtypeStruct((EMBED_DIM + 1, BATCH), jnp.float32),
        scratch_types=[
            pltpu.VMEM((N_CHUNKS, IDX_CHUNK), jnp.int32),
            pltpu.VMEM((2, IDX_PER_W), jnp.float32),
            pltpu.VMEM((EMBED_DIM + 1, B_PER_W), jnp.float32),
            pltpu.SemaphoreType.DMA((2,)),
        ],
        compiler_params=pltpu.CompilerParams(use_tc_tiling_on_sc=False),
    )
    def k(idx_hbm, table_hbm, out_hbm, idx_v, rows_v, pooled_v, sems):
        wid = lax.axis_index("s") * NC + lax.axis_index("c")
        pltpu.sync_copy(idx_hbm.at[wid], idx_v)

        def fire(d, slot):
            return [
                pltpu.async_copy(
                    table_hbm.at[d].at[idx_v.at[j]],
                    rows_v.at[slot, pl.ds(j * IDX_CHUNK, IDX_CHUNK)],
                    sems.at[slot],
                )
                for j in range(N_CHUNKS)
            ]

        pending = fire(0, 0)
        for d in range(EMBED_DIM):
            nxt = fire(d + 1, (d + 1) % 2) if d + 1 < EMBED_DIM else []
            for c in pending:
                c.wait()
            pending = nxt
            slot = d % 2
            for half in range(2):
                lo = half * 16
                acc = rows_v[slot, pl.ds(lo, 16)]
                for c2 in range(1, CTX):
                    acc = acc + rows_v[slot, pl.ds(c2 * B_PER_W + lo, 16)]
                pooled_v[d, pl.ds(lo, 16)] = acc
        ones = jnp.ones((16,), jnp.float32)
        pooled_v[EMBED_DIM, pl.ds(0, 16)] = ones
        pooled_v[EMBED_DIM, pl.ds(16, 16)] = ones
        pltpu.sync_copy(
            pooled_v, out_hbm.at[:, pl.ds(wid * B_PER_W, B_PER_W)]
        )

    return k(idx3, table_t)


def _mm_body(wt_ref, b_ref, xt_ref, o_hbm, scratch, sems):
    i = pl.program_id(0)
    slot = lax.rem(i, NBUF)

    @pl.when(i >= NBUF)
    def _wait_prev():
        j = i - NBUF
        pltpu.make_async_copy(
            scratch.at[slot], o_hbm.at[pl.ds(j * VT, VT)], sems.at[slot]
        ).wait()

    lhs = jnp.concatenate([wt_ref[...], b_ref[...]], axis=0)
    scratch[slot] = lax.dot_general(
        lhs, xt_ref[...],
        dimension_numbers=(((0,), (0,)), ((), ())),
        preferred_element_type=jnp.float32,
    )

    @pl.when(i < NSTEP - 1)
    def _start_full():
        pltpu.make_async_copy(
            scratch.at[slot], o_hbm.at[pl.ds(i * VT, VT)], sems.at[slot]
        ).start()

    @pl.when(i == NSTEP - 1)
    def _start_last():
        pltpu.make_async_copy(
            scratch.at[slot, pl.ds(0, LAST)],
            o_hbm.at[pl.ds(i * VT, LAST)],
            sems.at[slot],
        ).start()

    @pl.when(i == NSTEP - 1)
    def _drain():
        for k in range(NBUF - 1):
            j = i - (NBUF - 1) + k
            pltpu.make_async_copy(
                scratch.at[lax.rem(j, NBUF)],
                o_hbm.at[pl.ds(j * VT, VT)],
                sems.at[lax.rem(j, NBUF)],
            ).wait()
        pltpu.make_async_copy(
            scratch.at[slot, pl.ds(0, LAST)],
            o_hbm.at[pl.ds(i * VT, LAST)],
            sems.at[slot],
        ).wait()


def _project_tc(Wt, b2, xt_aug):
    out_t = pl.pallas_call(
        _mm_body,
        grid=(NSTEP,),
        in_specs=[
            pl.BlockSpec((EMBED_DIM, VT), lambda i: (0, i)),
            pl.BlockSpec((1, VT), lambda i: (0, i)),
            pl.BlockSpec((EMBED_DIM + 1, BATCH), lambda i: (0, 0)),
        ],
        out_specs=pl.BlockSpec(memory_space=pl.ANY),
        out_shape=jax.ShapeDtypeStruct((VOCAB, BATCH), jnp.float32),
        scratch_shapes=[
            pltpu.VMEM((NBUF, VT, BATCH), jnp.float32),
            pltpu.SemaphoreType.DMA((NBUF,)),
        ],
        compiler_params=pltpu.CompilerParams(
            vmem_limit_bytes=100 * 1024 * 1024,
        ),
    )(Wt, b2, xt_aug)
    return out_t


def kernel(inputs, embed_table, W, b):
    idx3 = (
        inputs.astype(jnp.int32)
        .reshape(NW, B_PER_W, CTX)
        .transpose(0, 2, 1)
        .reshape(NW, N_CHUNKS, IDX_CHUNK)
    )
    xt_aug = _pool_sc_t(idx3, embed_table.T)
    out_t = _project_tc(W.T, b.reshape(1, VOCAB), xt_aug)
    return out_t.T


# speedup vs baseline: 1.1308x; 1.0053x over previous
import functools

import jax
import jax.numpy as jnp
from jax import lax
from jax.experimental import pallas as pl
from jax.experimental.pallas import tpu as pltpu
from jax.experimental.pallas import tpu_sc as plsc

VOCAB = 100000
EMBED_DIM = 16
BATCH = 1024
CTX = 20

VT = 2048
NBUF = 6
NSTEP = (VOCAB + VT - 1) // VT
LAST = VOCAB - (NSTEP - 1) * VT

NC = 2
NS = 16
NW = NC * NS
B_PER_W = BATCH // NW
IDX_PER_W = B_PER_W * CTX
IDX_CHUNK = 128
N_CHUNKS = IDX_PER_W // IDX_CHUNK


def _pool_sc_t(idx3, table_t):
    mesh = plsc.VectorSubcoreMesh(core_axis_name="c", subcore_axis_name="s")

    @functools.partial(
        pl.kernel,
        mesh=mesh,
        out_type=jax.ShapeDtypeStruct((EMBED_DIM + 1, BATCH), jnp.float32),
        scratch_types=[
            pltpu.VMEM((N_CHUNKS, IDX_CHUNK), jnp.int32),
            pltpu.VMEM((2, IDX_PER_W), jnp.float32),
            pltpu.VMEM((EMBED_DIM + 1, B_PER_W), jnp.float32),
            pltpu.SemaphoreType.DMA((2,)),
        ],
        compiler_params=pltpu.CompilerParams(use_tc_tiling_on_sc=False),
    )
    def k(idx_hbm, table_hbm, out_hbm, idx_v, rows_v, pooled_v, sems):
        wid = lax.axis_index("s") * NC + lax.axis_index("c")
        pltpu.sync_copy(idx_hbm.at[wid], idx_v)

        def fire(d, slot):
            return [
                pltpu.async_copy(
                    table_hbm.at[d].at[idx_v.at[j]],
                    rows_v.at[slot, pl.ds(j * IDX_CHUNK, IDX_CHUNK)],
                    sems.at[slot],
                )
                for j in range(N_CHUNKS)
            ]

        pending = fire(0, 0)
        for d in range(EMBED_DIM):
            nxt = fire(d + 1, (d + 1) % 2) if d + 1 < EMBED_DIM else []
            for c in pending:
                c.wait()
            pending = nxt
            slot = d % 2
            for half in range(2):
                lo = half * 16
                acc = rows_v[slot, pl.ds(lo, 16)]
                for c2 in range(1, CTX):
                    acc = acc + rows_v[slot, pl.ds(c2 * B_PER_W + lo, 16)]
                pooled_v[d, pl.ds(lo, 16)] = acc
        ones = jnp.ones((16,), jnp.float32)
        pooled_v[EMBED_DIM, pl.ds(0, 16)] = ones
        pooled_v[EMBED_DIM, pl.ds(16, 16)] = ones
        pltpu.sync_copy(
            pooled_v, out_hbm.at[:, pl.ds(wid * B_PER_W, B_PER_W)]
        )

    return k(idx3, table_t)


def _mm_body(wt_ref, b_ref, xt_ref, o_hbm, scratch, sems):
    i = pl.program_id(0)
    slot = lax.rem(i, NBUF)

    @pl.when(i >= NBUF)
    def _wait_prev():
        j = i - NBUF
        pltpu.make_async_copy(
            scratch.at[slot], o_hbm.at[pl.ds(j * VT, VT)], sems.at[slot]
        ).wait()

    lhs = jnp.concatenate([wt_ref[...], b_ref[...]], axis=0)
    scratch[slot] = lax.dot_general(
        lhs, xt_ref[...],
        dimension_numbers=(((0,), (0,)), ((), ())),
        preferred_element_type=jnp.float32,
    )

    @pl.when(i < NSTEP - 1)
    def _start_full():
        pltpu.make_async_copy(
            scratch.at[slot], o_hbm.at[pl.ds(i * VT, VT)], sems.at[slot]
        ).start()

    @pl.when(i == NSTEP - 1)
    def _start_last():
        pltpu.make_async_copy(
            scratch.at[slot, pl.ds(0, LAST)],
            o_hbm.at[pl.ds(i * VT, LAST)],
            sems.at[slot],
        ).start()

    @pl.when(i == NSTEP - 1)
    def _drain():
        for k in range(NBUF - 1):
            j = i - (NBUF - 1) + k
            pltpu.make_async_copy(
                scratch.at[lax.rem(j, NBUF)],
                o_hbm.at[pl.ds(j * VT, VT)],
                sems.at[lax.rem(j, NBUF)],
            ).wait()
        pltpu.make_async_copy(
            scratch.at[slot, pl.ds(0, LAST)],
            o_hbm.at[pl.ds(i * VT, LAST)],
            sems.at[slot],
        ).wait()


def _project_tc(Wt, b2, xt_aug):
    out_t = pl.pallas_call(
        _mm_body,
        grid=(NSTEP,),
        in_specs=[
            pl.BlockSpec((EMBED_DIM, VT), lambda i: (0, i)),
            pl.BlockSpec((1, VT), lambda i: (0, i)),
            pl.BlockSpec((EMBED_DIM + 1, BATCH), lambda i: (0, 0)),
        ],
        out_specs=pl.BlockSpec(memory_space=pl.ANY),
        out_shape=jax.ShapeDtypeStruct((VOCAB, BATCH), jnp.float32),
        scratch_shapes=[
            pltpu.VMEM((NBUF, VT, BATCH), jnp.float32),
            pltpu.SemaphoreType.DMA((NBUF,)),
        ],
        compiler_params=pltpu.CompilerParams(
            vmem_limit_bytes=100 * 1024 * 1024,
        ),
    )(Wt, b2, xt_aug)
    return out_t


def kernel(inputs, embed_table, W, b):
    idx3 = (
        inputs.astype(jnp.int32)
        .reshape(NW, B_PER_W, CTX)
        .transpose(0, 2, 1)
        .reshape(NW, N_CHUNKS, IDX_CHUNK)
    )
    xt_aug = _pool_sc_t(idx3, embed_table.T)
    out_t = _project_tc(W.T, b.reshape(1, VOCAB), xt_aug)
    return out_t.T


# speedup vs baseline: 1.1485x; 1.0156x over previous
import functools

import jax
import jax.numpy as jnp
from jax import lax
from jax.experimental import pallas as pl
from jax.experimental.pallas import tpu as pltpu
from jax.experimental.pallas import tpu_sc as plsc

VOCAB = 100000
EMBED_DIM = 16
BATCH = 1024
CTX = 20

VT = 2048
NBUF = 6
NSTEP = (VOCAB + VT - 1) // VT
LAST = VOCAB - (NSTEP - 1) * VT

NC = 2
NS = 16
NW = NC * NS
B_PER_W = BATCH // NW
IDX_PER_W = B_PER_W * CTX
IDX_CHUNK = 128
N_CHUNKS = IDX_PER_W // IDX_CHUNK


def _pool_sc_t(idx3, table_t):
    mesh = plsc.VectorSubcoreMesh(core_axis_name="c", subcore_axis_name="s")

    @functools.partial(
        pl.kernel,
        mesh=mesh,
        out_type=jax.ShapeDtypeStruct((EMBED_DIM + 1, BATCH), jnp.float32),
        scratch_types=[
            pltpu.VMEM((N_CHUNKS, IDX_CHUNK), jnp.int32),
            pltpu.VMEM((4, IDX_PER_W), jnp.float32),
            pltpu.VMEM((EMBED_DIM + 1, B_PER_W), jnp.float32),
            pltpu.SemaphoreType.DMA((4,)),
        ],
        compiler_params=pltpu.CompilerParams(use_tc_tiling_on_sc=False),
    )
    def k(idx_hbm, table_hbm, out_hbm, idx_v, rows_v, pooled_v, sems):
        wid = lax.axis_index("s") * NC + lax.axis_index("c")
        pltpu.sync_copy(idx_hbm.at[wid], idx_v)

        def fire(d, slot):
            return [
                pltpu.async_copy(
                    table_hbm.at[d].at[idx_v.at[j]],
                    rows_v.at[slot, pl.ds(j * IDX_CHUNK, IDX_CHUNK)],
                    sems.at[slot],
                )
                for j in range(N_CHUNKS)
            ]

        pending = {d: fire(d, d % 4) for d in range(3)}
        for d in range(EMBED_DIM):
            if d + 3 < EMBED_DIM:
                pending[d + 3] = fire(d + 3, (d + 3) % 4)
            for c in pending.pop(d):
                c.wait()
            slot = d % 4
            for half in range(2):
                lo = half * 16
                acc = rows_v[slot, pl.ds(lo, 16)]
                for c2 in range(1, CTX):
                    acc = acc + rows_v[slot, pl.ds(c2 * B_PER_W + lo, 16)]
                pooled_v[d, pl.ds(lo, 16)] = acc
        ones = jnp.ones((16,), jnp.float32)
        pooled_v[EMBED_DIM, pl.ds(0, 16)] = ones
        pooled_v[EMBED_DIM, pl.ds(16, 16)] = ones
        pltpu.sync_copy(
            pooled_v, out_hbm.at[:, pl.ds(wid * B_PER_W, B_PER_W)]
        )

    return k(idx3, table_t)


def _mm_body(wt_ref, b_ref, xt_ref, o_hbm, scratch, sems):
    i = pl.program_id(0)
    slot = lax.rem(i, NBUF)

    @pl.when(i >= NBUF)
    def _wait_prev():
        j = i - NBUF
        pltpu.make_async_copy(
            scratch.at[slot], o_hbm.at[pl.ds(j * VT, VT)], sems.at[slot]
        ).wait()

    lhs = jnp.concatenate([wt_ref[...], b_ref[...]], axis=0)
    scratch[slot] = lax.dot_general(
        lhs, xt_ref[...],
        dimension_numbers=(((0,), (0,)), ((), ())),
        preferred_element_type=jnp.float32,
    )

    @pl.when(i < NSTEP - 1)
    def _start_full():
        pltpu.make_async_copy(
            scratch.at[slot], o_hbm.at[pl.ds(i * VT, VT)], sems.at[slot]
        ).start()

    @pl.when(i == NSTEP - 1)
    def _start_last():
        pltpu.make_async_copy(
            scratch.at[slot, pl.ds(0, LAST)],
            o_hbm.at[pl.ds(i * VT, LAST)],
            sems.at[slot],
        ).start()

    @pl.when(i == NSTEP - 1)
    def _drain():
        for k in range(NBUF - 1):
            j = i - (NBUF - 1) + k
            pltpu.make_async_copy(
                scratch.at[lax.rem(j, NBUF)],
                o_hbm.at[pl.ds(j * VT, VT)],
                sems.at[lax.rem(j, NBUF)],
            ).wait()
        pltpu.make_async_copy(
            scratch.at[slot, pl.ds(0, LAST)],
            o_hbm.at[pl.ds(i * VT, LAST)],
            sems.at[slot],
        ).wait()


def _project_tc(Wt, b2, xt_aug):
    out_t = pl.pallas_call(
        _mm_body,
        grid=(NSTEP,),
        in_specs=[
            pl.BlockSpec((EMBED_DIM, VT), lambda i: (0, i)),
            pl.BlockSpec((1, VT), lambda i: (0, i)),
            pl.BlockSpec((EMBED_DIM + 1, BATCH), lambda i: (0, 0)),
        ],
        out_specs=pl.BlockSpec(memory_space=pl.ANY),
        out_shape=jax.ShapeDtypeStruct((VOCAB, BATCH), jnp.float32),
        scratch_shapes=[
            pltpu.VMEM((NBUF, VT, BATCH), jnp.float32),
            pltpu.SemaphoreType.DMA((NBUF,)),
        ],
        compiler_params=pltpu.CompilerParams(
            vmem_limit_bytes=100 * 1024 * 1024,
        ),
    )(Wt, b2, xt_aug)
    return out_t


def kernel(inputs, embed_table, W, b):
    idx3 = (
        inputs.astype(jnp.int32)
        .reshape(NW, B_PER_W, CTX)
        .transpose(0, 2, 1)
        .reshape(NW, N_CHUNKS, IDX_CHUNK)
    )
    xt_aug = _pool_sc_t(idx3, embed_table.T)
    out_t = _project_tc(W.T, b.reshape(1, VOCAB), xt_aug)
    return out_t.T
